# Initial kernel scaffold; baseline (speedup 1.0000x reference)
import jax
import jax.numpy as jnp
from jax.experimental import pallas as pl


def kernel(user_ids, s_item_ids, t_item_ids, edges_s, edges_t, feats, params):
    raise NotImplementedError("write your pallas kernel here")


# speedup vs baseline: 13.9637x; 13.9637x over previous
import functools

import jax
import jax.numpy as jnp
from jax import lax
from jax.experimental import pallas as pl
from jax.experimental.pallas import tpu as pltpu
from jax.experimental.pallas import tpu_sc as plsc

U, NI = 10000, 20000
UP, VP = 10240, 20480
D, E, B = 64, 320000, 1024
NT = 16
PER_TILE = E // NT
C = 125
NCH = PER_TILE // C
URT = UP // NT
VRT = VP // NT
BW = B // 32
DW = 16


def _degrees_body(idx_u_hbm, idx_i_hbm, ones_hbm, zcol_hbm,
                  du_hbm, dv_hbm, idx, ones_v, acc_u, acc_v, sem):
    c = lax.axis_index("c")
    s = lax.axis_index("s")
    pltpu.sync_copy(zcol_hbm.at[pl.ds(0, URT)], acc_u.at[pl.ds(URT * s, URT)])
    pltpu.sync_copy(zcol_hbm, acc_v.at[pl.ds(VRT * s, VRT)])
    pltpu.sync_copy(ones_hbm, ones_v)
    plsc.subcore_barrier()

    pltpu.sync_copy(idx_u_hbm.at[c, s], idx)

    def body_u(j, carry):
        pltpu.sync_copy(ones_v, acc_u.at[idx.at[j]], add=True)
        return carry
    lax.fori_loop(0, NCH, body_u, 0)

    pltpu.sync_copy(idx_i_hbm.at[c, s], idx)

    def body_i(j, carry):
        pltpu.sync_copy(ones_v, acc_v.at[idx.at[j]], add=True)
        return carry
    lax.fori_loop(0, NCH, body_i, 0)

    plsc.subcore_barrier()
    pltpu.sync_copy(acc_u.at[pl.ds(URT * s, URT)],
                    du_hbm.at[c, pl.ds(URT * s, URT)])
    pltpu.sync_copy(acc_v.at[pl.ds(VRT * s, VRT)],
                    dv_hbm.at[c, pl.ds(VRT * s, VRT)])


def _pass_body(src_hbm, ig, isc, zrows_hbm, out_hbm,
               idx_g, idx_s, rows, acc, sem, *, nrt, coh):
    c = lax.axis_index("c")
    s = lax.axis_index("s")
    pltpu.sync_copy(zrows_hbm.at[pl.ds(0, nrt)], acc.at[pl.ds(nrt * s, nrt)])
    plsc.subcore_barrier()

    for b in range(NCH // coh):
        pltpu.sync_copy(ig.at[c, s, pl.ds(b * coh, coh)], idx_g)
        pltpu.sync_copy(isc.at[c, s, pl.ds(b * coh, coh)], idx_s)

        def body(j, carry):
            pltpu.async_copy(src_hbm.at[idx_g.at[j]], rows, sem).wait()
            pltpu.sync_copy(rows, acc.at[idx_s.at[j]], add=True)
            return carry
        lax.fori_loop(0, coh, body, 0)

    plsc.subcore_barrier()
    pltpu.sync_copy(acc.at[pl.ds(nrt * s, nrt)],
                    out_hbm.at[c, pl.ds(nrt * s, nrt)])


_GATHER_WIDTHS = (64, 64, 64, 64, 64, 64, DW, DW, 384, 384,
                  64, 64, 64, 64, 64, 64, DW, DW, 384, 384, 512, 512)


def _gather_body(*refs):
    tables = refs[:22]
    ids_flat = refs[22]
    outs = refs[23:45]
    idbuf, b64, b384, b512, b16, sem = refs[45:]
    bufs = {64: b64, 384: b384, 512: b512, DW: b16}
    c = lax.axis_index("c")
    s = lax.axis_index("s")
    wid = s * 2 + c
    base = wid * BW
    for r in range(22):
        buf = bufs[_GATHER_WIDTHS[r]]
        pltpu.sync_copy(ids_flat.at[pl.ds(r * B + base, BW)], idbuf)
        pltpu.async_copy(tables[r].at[idbuf], buf, sem).wait()
        pltpu.sync_copy(buf, outs[r].at[pl.ds(base, BW)])


@functools.lru_cache(maxsize=1)
def _sc_kernels():
    mesh = plsc.VectorSubcoreMesh(core_axis_name="c", subcore_axis_name="s")
    cp = pltpu.CompilerParams(use_tc_tiling_on_sc=False)
    degrees = pl.kernel(
        _degrees_body,
        out_type=(jax.ShapeDtypeStruct((2, UP, DW), jnp.float32),
                  jax.ShapeDtypeStruct((2, VP, DW), jnp.float32)),
        mesh=mesh,
        compiler_params=cp,
        scratch_types=(pltpu.VMEM((NCH, C), jnp.int32),
                       pltpu.VMEM((C, DW), jnp.float32),
                       pltpu.VMEM_SHARED((UP, DW), jnp.float32),
                       pltpu.VMEM_SHARED((VP, DW), jnp.float32),
                       pltpu.SemaphoreType.DMA),
    )
    def make_pass(nrows, nrt, coh):
        return pl.kernel(
            functools.partial(_pass_body, nrt=nrt, coh=coh),
            out_type=jax.ShapeDtypeStruct((2, nrows, D), jnp.float32),
            mesh=mesh,
            compiler_params=cp,
            scratch_types=(pltpu.VMEM((coh, C), jnp.int32),
                           pltpu.VMEM((coh, C), jnp.int32),
                           pltpu.VMEM((C, D), jnp.float32),
                           pltpu.VMEM_SHARED((nrows, D), jnp.float32),
                           pltpu.SemaphoreType.DMA),
        )
    pass_u = make_pass(UP, URT, NCH)
    pass_v = make_pass(VP, VRT, 40)
    batch_gather = pl.kernel(
        _gather_body,
        out_type=tuple(jax.ShapeDtypeStruct((B, w), jnp.float32)
                       for w in _GATHER_WIDTHS),
        mesh=mesh,
        compiler_params=cp,
        scratch_types=(pltpu.VMEM((BW,), jnp.int32),
                       pltpu.VMEM((BW, 64), jnp.float32),
                       pltpu.VMEM((BW, 384), jnp.float32),
                       pltpu.VMEM((BW, 512), jnp.float32),
                       pltpu.VMEM((BW, DW), jnp.float32),
                       pltpu.SemaphoreType.DMA),
    )
    return degrees, pass_u, pass_v, batch_gather


def _scale_kernel(d_ref, t_ref, o_ref, *, inv):
    d = jnp.maximum(d_ref[...][:, :1], 1.0)
    scl = (1.0 / d) if inv else lax.rsqrt(d)
    o_ref[...] = t_ref[...] * scl


def _scale(table, deg, inv, br=1280):
    n = table.shape[0]
    return pl.pallas_call(
        functools.partial(_scale_kernel, inv=inv),
        grid=(n // br,),
        in_specs=[pl.BlockSpec((br, DW), lambda i: (i, 0)),
                  pl.BlockSpec((br, D), lambda i: (i, 0))],
        out_specs=pl.BlockSpec((br, D), lambda i: (i, 0)),
        out_shape=jax.ShapeDtypeStruct((n, D), jnp.float32),
    )(deg, table)


def _head_kernel(*refs):
    (u0s, u0t, t1us, t1ut, t2us, t2ut, dus, dut, revs, revt,
     v0s, v0t, t1vs, t1vt, t2vs, t2vt, dis, dit, stext, ttext, svis, tvis,
     W_rev_s, b_rev_s, W_rev_t, b_rev_t, W_text_s, b_text_s, W_text_t,
     b_text_t, W_vis_s, b_vis_s, W_vis_t, b_vis_t, W_sh, b_sh, W_ps, b_ps,
     W_pt, b_pt, W_cat_s, b_cat_s, W_cat_t, b_cat_t, W1_s, b1_s, W2_s, b2_s,
     W3_s, b3_s, W1_t, b1_t, W2_t, b2_t, W3_t, b3_t, out) = refs

    relu = jax.nn.relu

    ru_s = lax.rsqrt(jnp.maximum(dus[...][:, :1], 1.0))
    ru_t = lax.rsqrt(jnp.maximum(dut[...][:, :1], 1.0))
    ri_s = lax.rsqrt(jnp.maximum(dis[...][:, :1], 1.0))
    ri_t = lax.rsqrt(jnp.maximum(dit[...][:, :1], 1.0))

    su_g = (u0s[...] + ru_s * (t1us[...] + t2us[...])) * (1.0 / 3.0)
    tu_g = (u0t[...] + ru_t * (t1ut[...] + t2ut[...])) * (1.0 / 3.0)
    sv_g = (v0s[...] + ri_s * (t1vs[...] + t2vs[...])) * (1.0 / 3.0)
    tv_g = (v0t[...] + ri_t * (t1vt[...] + t2vt[...])) * (1.0 / 3.0)

    su = su_g + relu(jnp.dot(revs[...], W_rev_s[...]) + b_rev_s[...])
    tu = tu_g + relu(jnp.dot(revt[...], W_rev_t[...]) + b_rev_t[...])
    si = (sv_g + relu(jnp.dot(stext[...], W_text_s[...]) + b_text_s[...])
          + relu(jnp.dot(svis[...], W_vis_s[...]) + b_vis_s[...]))
    ti = (tv_g + relu(jnp.dot(ttext[...], W_text_t[...]) + b_text_t[...])
          + relu(jnp.dot(tvis[...], W_vis_t[...]) + b_vis_t[...]))

    shared = relu(jnp.dot(jnp.concatenate([su, tu], 1), W_sh[...]) + b_sh[...])
    ps = relu(jnp.dot(su, W_ps[...]) + b_ps[...])
    pt = relu(jnp.dot(tu, W_pt[...]) + b_pt[...])
    s_fu = jnp.dot(jnp.concatenate([shared, ps], 1), W_cat_s[...]) + b_cat_s[...]
    t_fu = jnp.dot(jnp.concatenate([shared, pt], 1), W_cat_t[...]) + b_cat_t[...]

    def bn(h):
        m = jnp.mean(h, axis=0, keepdims=True)
        v = jnp.mean((h - m) ** 2, axis=0, keepdims=True)
        return (h - m) / jnp.sqrt(v + 1e-5)

    def head(uu, ii, W1, b1, W2, b2, W3, b3):
        h = jnp.dot(jnp.concatenate([uu, ii], 1), W1[...]) + b1[...]
        h = relu(bn(h))
        h = jnp.dot(h, W2[...]) + b2[...]
        h = relu(bn(h))
        return jnp.sum(h * W3[...], axis=1, keepdims=True) + b3[...]

    out_s = head(s_fu, si, W1_s, b1_s, W2_s, b2_s, W3_s, b3_s)
    out_t = head(t_fu, ti, W1_t, b1_t, W2_t, b2_t, W3_t, b3_t)
    out[...] = jnp.concatenate([out_s, out_t], 1)


def _run_head(batch_arrays, p):
    wlist = [
        p["W_rev_s"], p["b_rev_s"], p["W_rev_t"], p["b_rev_t"],
        p["W_text_s"], p["b_text_s"], p["W_text_t"], p["b_text_t"],
        p["W_vis_s"], p["b_vis_s"], p["W_vis_t"], p["b_vis_t"],
        p["W_sh"], p["b_sh"], p["W_ps"], p["b_ps"], p["W_pt"], p["b_pt"],
        p["W_cat_s"], p["b_cat_s"], p["W_cat_t"], p["b_cat_t"],
        p["W1_s"], p["b1_s"], p["W2_s"], p["b2_s"], p["W3_s"], p["b3_s"],
        p["W1_t"], p["b1_t"], p["W2_t"], p["b2_t"], p["W3_t"], p["b3_t"],
    ]
    wlist = [w.reshape(1, -1) if w.ndim == 1 else w for w in wlist]
    wlist[26] = wlist[26].reshape(1, -1)
    wlist[32] = wlist[32].reshape(1, -1)
    args = list(batch_arrays) + wlist
    return pl.pallas_call(
        _head_kernel,
        out_shape=jax.ShapeDtypeStruct((B, 2), jnp.float32),
    )(*args)


def kernel(user_ids, s_item_ids, t_item_ids, edges_s, edges_t, feats, params):
    p = params
    f = feats

    def rsh(x):
        return x.reshape(NT, NCH, C)

    eu_s, ei_s = edges_s[0], edges_s[1]
    eu_t, ei_t = edges_t[0], edges_t[1]
    ig1 = jnp.stack([rsh(ei_s), rsh(ei_t + VP)])
    is1 = jnp.stack([rsh(eu_s), rsh(eu_t)])
    ig2 = jnp.stack([rsh(eu_s), rsh(eu_t + UP)])
    is2 = jnp.stack([rsh(ei_s), rsh(ei_t)])

    ones_col = jnp.ones((C, DW), jnp.float32)
    zcol = jnp.zeros((VRT, DW), jnp.float32)
    zrows = jnp.zeros((VRT, D), jnp.float32)

    _degrees, _pass_u, _pass_v, _batch_gather = _sc_kernels()
    du, dv = _degrees(is1, is2, ones_col, zcol)
    du2 = du.reshape(2 * UP, DW)
    dv2 = dv.reshape(2 * VP, DW)

    pad_u = jnp.zeros((UP - U, D), jnp.float32)
    pad_v = jnp.zeros((VP - NI, D), jnp.float32)
    u0 = jnp.concatenate([p["s_u_emb"], pad_u, p["t_u_emb"], pad_u], 0)
    v0 = jnp.concatenate([p["s_v_emb"], pad_v, p["t_v_emb"], pad_v], 0)

    src_u = _scale(u0, du2, False)
    src_v = _scale(v0, dv2, False)
    t1u = _pass_u(src_v, ig1, is1, zrows).reshape(2 * UP, D)
    t1v = _pass_v(src_u, ig2, is2, zrows).reshape(2 * VP, D)
    y = _scale(t1u, du2, True)
    x = _scale(t1v, dv2, True)
    t2u = _pass_u(x, ig1, is1, zrows).reshape(2 * UP, D)
    t2v = _pass_v(y, ig2, is2, zrows).reshape(2 * VP, D)

    uid = user_ids
    sid = s_item_ids
    tid = t_item_ids
    ids_flat = jnp.stack([
        uid, uid, uid, uid + UP, uid, uid + UP, uid, uid + UP, uid, uid,
        sid, tid, sid, tid + VP, sid, tid + VP, sid, tid + VP,
        sid, tid, sid, tid,
    ]).astype(jnp.int32).reshape(-1)
    tables = (p["s_u_emb"], p["t_u_emb"], t1u, t1u, t2u, t2u, du2, du2,
              f["s_rev"], f["t_rev"],
              p["s_v_emb"], p["t_v_emb"], t1v, t1v, t2v, t2v, dv2, dv2,
              f["s_text"], f["t_text"], f["s_vis"], f["t_vis"])
    gathered = _batch_gather(*tables, ids_flat)

    return _run_head(gathered, p)


# speedup vs baseline: 20.0179x; 1.4336x over previous
import functools

import jax
import jax.numpy as jnp
from jax import lax
from jax.experimental import pallas as pl
from jax.experimental.pallas import tpu as pltpu
from jax.experimental.pallas import tpu_sc as plsc

U, NI = 10000, 20000
UP, VP = 10240, 20480
D, E, B = 64, 320000, 1024
NT = 16
PER_TILE = E // NT
C = 125
NCH = PER_TILE // C
URT = UP // NT
VRT = VP // NT
BW = B // 32
DW = 16


def _degrees_body(idx_u_hbm, idx_i_hbm, ones_hbm, zcol_hbm,
                  du_hbm, dv_hbm, idx, ones_v, acc_u, acc_v, sem):
    c = lax.axis_index("c")
    s = lax.axis_index("s")
    pltpu.sync_copy(zcol_hbm.at[pl.ds(0, URT)], acc_u.at[pl.ds(URT * s, URT)])
    pltpu.sync_copy(zcol_hbm, acc_v.at[pl.ds(VRT * s, VRT)])
    pltpu.sync_copy(ones_hbm, ones_v)
    plsc.subcore_barrier()

    def burst(acc):
        def body(t, carry):
            for q in range(8):
                pltpu.async_copy(ones_v, acc.at[idx.at[8 * t + q]],
                                 sem, add=True)
            for q in range(8):
                pltpu.make_async_copy(ones_v, acc.at[idx.at[8 * t + q]],
                                      sem).wait()
            return carry
        lax.fori_loop(0, NCH // 8, body, 0)

    pltpu.sync_copy(idx_u_hbm.at[c, s], idx)
    burst(acc_u)
    pltpu.sync_copy(idx_i_hbm.at[c, s], idx)
    burst(acc_v)

    plsc.subcore_barrier()
    pltpu.sync_copy(acc_u.at[pl.ds(URT * s, URT)],
                    du_hbm.at[c, pl.ds(URT * s, URT)])
    pltpu.sync_copy(acc_v.at[pl.ds(VRT * s, VRT)],
                    dv_hbm.at[c, pl.ds(VRT * s, VRT)])


def _pass_body(src_hbm, ig, isc, zrows_hbm, out_hbm,
               idx_g, idx_s, rows0, rows1, acc, semg0, semg1, *, nrt, coh):
    c = lax.axis_index("c")
    s = lax.axis_index("s")
    pltpu.sync_copy(zrows_hbm.at[pl.ds(0, nrt)], acc.at[pl.ds(nrt * s, nrt)])
    plsc.subcore_barrier()

    rows = (rows0, rows1)
    semg = (semg0, semg1)
    for b in range(NCH // coh):
        pltpu.sync_copy(ig.at[c, s, pl.ds(b * coh, coh)], idx_g)
        pltpu.sync_copy(isc.at[c, s, pl.ds(b * coh, coh)], idx_s)
        pltpu.async_copy(src_hbm.at[idx_g.at[0]], rows0, semg0)
        pltpu.async_copy(src_hbm.at[idx_g.at[1]], rows1, semg1)

        def step(t, carry):
            for q in range(2):
                j = 2 * t + q
                pltpu.make_async_copy(src_hbm.at[idx_g.at[j]],
                                      rows[q], semg[q]).wait()
                pltpu.sync_copy(rows[q], acc.at[idx_s.at[j]], add=True)
                pltpu.async_copy(src_hbm.at[idx_g.at[j + 2]], rows[q], semg[q])
            return carry
        lax.fori_loop(0, coh // 2 - 1, step, 0)

        for q in range(2):
            j = coh - 2 + q
            pltpu.make_async_copy(src_hbm.at[idx_g.at[j]],
                                  rows[q], semg[q]).wait()
            pltpu.sync_copy(rows[q], acc.at[idx_s.at[j]], add=True)

    plsc.subcore_barrier()
    pltpu.sync_copy(acc.at[pl.ds(nrt * s, nrt)],
                    out_hbm.at[c, pl.ds(nrt * s, nrt)])


_GATHER_WIDTHS = (64, 64, 64, 64, 64, 64, DW, DW, 384, 384,
                  64, 64, 64, 64, 64, 64, DW, DW, 384, 384, 512, 512)


def _gather_body(*refs):
    tables = refs[:22]
    ids_flat = refs[22]
    outs = refs[23:45]
    idbuf, b64, b384, b512, b16, sem = refs[45:]
    bufs = {64: b64, 384: b384, 512: b512, DW: b16}
    c = lax.axis_index("c")
    s = lax.axis_index("s")
    wid = s * 2 + c
    base = wid * BW
    for r in range(22):
        buf = bufs[_GATHER_WIDTHS[r]]
        pltpu.sync_copy(ids_flat.at[pl.ds(r * B + base, BW)], idbuf)
        pltpu.async_copy(tables[r].at[idbuf], buf, sem).wait()
        pltpu.sync_copy(buf, outs[r].at[pl.ds(base, BW)])


@functools.lru_cache(maxsize=1)
def _sc_kernels():
    mesh = plsc.VectorSubcoreMesh(core_axis_name="c", subcore_axis_name="s")
    cp = pltpu.CompilerParams(use_tc_tiling_on_sc=False)
    degrees = pl.kernel(
        _degrees_body,
        out_type=(jax.ShapeDtypeStruct((2, UP, DW), jnp.float32),
                  jax.ShapeDtypeStruct((2, VP, DW), jnp.float32)),
        mesh=mesh,
        compiler_params=cp,
        scratch_types=(pltpu.VMEM((NCH, C), jnp.int32),
                       pltpu.VMEM((C, DW), jnp.float32),
                       pltpu.VMEM_SHARED((UP, DW), jnp.float32),
                       pltpu.VMEM_SHARED((VP, DW), jnp.float32),
                       pltpu.SemaphoreType.DMA),
    )
    def make_pass(nrows, nrt, coh):
        return pl.kernel(
            functools.partial(_pass_body, nrt=nrt, coh=coh),
            out_type=jax.ShapeDtypeStruct((2, nrows, D), jnp.float32),
            mesh=mesh,
            compiler_params=cp,
            scratch_types=(pltpu.VMEM((coh, C), jnp.int32),
                           pltpu.VMEM((coh, C), jnp.int32),
                           pltpu.VMEM((C, D), jnp.float32),
                           pltpu.VMEM((C, D), jnp.float32),
                           pltpu.VMEM_SHARED((nrows, D), jnp.float32),
                           pltpu.SemaphoreType.DMA,
                           pltpu.SemaphoreType.DMA),
        )
    pass_u = make_pass(UP, URT, NCH)
    pass_v = make_pass(VP, VRT, 80)
    batch_gather = pl.kernel(
        _gather_body,
        out_type=tuple(jax.ShapeDtypeStruct((B, w), jnp.float32)
                       for w in _GATHER_WIDTHS),
        mesh=mesh,
        compiler_params=cp,
        scratch_types=(pltpu.VMEM((BW,), jnp.int32),
                       pltpu.VMEM((BW, 64), jnp.float32),
                       pltpu.VMEM((BW, 384), jnp.float32),
                       pltpu.VMEM((BW, 512), jnp.float32),
                       pltpu.VMEM((BW, DW), jnp.float32),
                       pltpu.SemaphoreType.DMA),
    )
    return degrees, pass_u, pass_v, batch_gather


def _scale_kernel(d_ref, t_ref, o_ref, *, inv):
    d = jnp.maximum(d_ref[...][:, :1], 1.0)
    scl = (1.0 / d) if inv else lax.rsqrt(d)
    o_ref[...] = t_ref[...] * scl


def _scale(table, deg, inv, br=1280):
    n = table.shape[0]
    return pl.pallas_call(
        functools.partial(_scale_kernel, inv=inv),
        grid=(n // br,),
        in_specs=[pl.BlockSpec((br, DW), lambda i: (i, 0)),
                  pl.BlockSpec((br, D), lambda i: (i, 0))],
        out_specs=pl.BlockSpec((br, D), lambda i: (i, 0)),
        out_shape=jax.ShapeDtypeStruct((n, D), jnp.float32),
    )(deg, table)


def _head_kernel(*refs):
    (u0s, u0t, t1us, t1ut, t2us, t2ut, dus, dut, revs, revt,
     v0s, v0t, t1vs, t1vt, t2vs, t2vt, dis, dit, stext, ttext, svis, tvis,
     W_rev_s, b_rev_s, W_rev_t, b_rev_t, W_text_s, b_text_s, W_text_t,
     b_text_t, W_vis_s, b_vis_s, W_vis_t, b_vis_t, W_sh, b_sh, W_ps, b_ps,
     W_pt, b_pt, W_cat_s, b_cat_s, W_cat_t, b_cat_t, W1_s, b1_s, W2_s, b2_s,
     W3_s, b3_s, W1_t, b1_t, W2_t, b2_t, W3_t, b3_t, out) = refs

    relu = jax.nn.relu

    ru_s = lax.rsqrt(jnp.maximum(dus[...][:, :1], 1.0))
    ru_t = lax.rsqrt(jnp.maximum(dut[...][:, :1], 1.0))
    ri_s = lax.rsqrt(jnp.maximum(dis[...][:, :1], 1.0))
    ri_t = lax.rsqrt(jnp.maximum(dit[...][:, :1], 1.0))

    su_g = (u0s[...] + ru_s * (t1us[...] + t2us[...])) * (1.0 / 3.0)
    tu_g = (u0t[...] + ru_t * (t1ut[...] + t2ut[...])) * (1.0 / 3.0)
    sv_g = (v0s[...] + ri_s * (t1vs[...] + t2vs[...])) * (1.0 / 3.0)
    tv_g = (v0t[...] + ri_t * (t1vt[...] + t2vt[...])) * (1.0 / 3.0)

    su = su_g + relu(jnp.dot(revs[...], W_rev_s[...]) + b_rev_s[...])
    tu = tu_g + relu(jnp.dot(revt[...], W_rev_t[...]) + b_rev_t[...])
    si = (sv_g + relu(jnp.dot(stext[...], W_text_s[...]) + b_text_s[...])
          + relu(jnp.dot(svis[...], W_vis_s[...]) + b_vis_s[...]))
    ti = (tv_g + relu(jnp.dot(ttext[...], W_text_t[...]) + b_text_t[...])
          + relu(jnp.dot(tvis[...], W_vis_t[...]) + b_vis_t[...]))

    shared = relu(jnp.dot(jnp.concatenate([su, tu], 1), W_sh[...]) + b_sh[...])
    ps = relu(jnp.dot(su, W_ps[...]) + b_ps[...])
    pt = relu(jnp.dot(tu, W_pt[...]) + b_pt[...])
    s_fu = jnp.dot(jnp.concatenate([shared, ps], 1), W_cat_s[...]) + b_cat_s[...]
    t_fu = jnp.dot(jnp.concatenate([shared, pt], 1), W_cat_t[...]) + b_cat_t[...]

    def bn(h):
        m = jnp.mean(h, axis=0, keepdims=True)
        v = jnp.mean((h - m) ** 2, axis=0, keepdims=True)
        return (h - m) / jnp.sqrt(v + 1e-5)

    def head(uu, ii, W1, b1, W2, b2, W3, b3):
        h = jnp.dot(jnp.concatenate([uu, ii], 1), W1[...]) + b1[...]
        h = relu(bn(h))
        h = jnp.dot(h, W2[...]) + b2[...]
        h = relu(bn(h))
        return jnp.sum(h * W3[...], axis=1, keepdims=True) + b3[...]

    out_s = head(s_fu, si, W1_s, b1_s, W2_s, b2_s, W3_s, b3_s)
    out_t = head(t_fu, ti, W1_t, b1_t, W2_t, b2_t, W3_t, b3_t)
    out[...] = jnp.concatenate([out_s, out_t], 1)


def _run_head(batch_arrays, p):
    wlist = [
        p["W_rev_s"], p["b_rev_s"], p["W_rev_t"], p["b_rev_t"],
        p["W_text_s"], p["b_text_s"], p["W_text_t"], p["b_text_t"],
        p["W_vis_s"], p["b_vis_s"], p["W_vis_t"], p["b_vis_t"],
        p["W_sh"], p["b_sh"], p["W_ps"], p["b_ps"], p["W_pt"], p["b_pt"],
        p["W_cat_s"], p["b_cat_s"], p["W_cat_t"], p["b_cat_t"],
        p["W1_s"], p["b1_s"], p["W2_s"], p["b2_s"], p["W3_s"], p["b3_s"],
        p["W1_t"], p["b1_t"], p["W2_t"], p["b2_t"], p["W3_t"], p["b3_t"],
    ]
    wlist = [w.reshape(1, -1) if w.ndim == 1 else w for w in wlist]
    wlist[26] = wlist[26].reshape(1, -1)
    wlist[32] = wlist[32].reshape(1, -1)
    args = list(batch_arrays) + wlist
    return pl.pallas_call(
        _head_kernel,
        out_shape=jax.ShapeDtypeStruct((B, 2), jnp.float32),
    )(*args)


def kernel(user_ids, s_item_ids, t_item_ids, edges_s, edges_t, feats, params):
    p = params
    f = feats

    def rsh(x):
        return x.reshape(NT, NCH, C)

    eu_s, ei_s = edges_s[0], edges_s[1]
    eu_t, ei_t = edges_t[0], edges_t[1]
    ig1 = jnp.stack([rsh(ei_s), rsh(ei_t + VP)])
    is1 = jnp.stack([rsh(eu_s), rsh(eu_t)])
    ig2 = jnp.stack([rsh(eu_s), rsh(eu_t + UP)])
    is2 = jnp.stack([rsh(ei_s), rsh(ei_t)])

    ones_col = jnp.ones((C, DW), jnp.float32)
    zcol = jnp.zeros((VRT, DW), jnp.float32)
    zrows = jnp.zeros((VRT, D), jnp.float32)

    _degrees, _pass_u, _pass_v, _batch_gather = _sc_kernels()
    du, dv = _degrees(is1, is2, ones_col, zcol)
    du2 = du.reshape(2 * UP, DW)
    dv2 = dv.reshape(2 * VP, DW)

    pad_u = jnp.zeros((UP - U, D), jnp.float32)
    pad_v = jnp.zeros((VP - NI, D), jnp.float32)
    u0 = jnp.concatenate([p["s_u_emb"], pad_u, p["t_u_emb"], pad_u], 0)
    v0 = jnp.concatenate([p["s_v_emb"], pad_v, p["t_v_emb"], pad_v], 0)

    src_u = _scale(u0, du2, False)
    src_v = _scale(v0, dv2, False)
    t1u = _pass_u(src_v, ig1, is1, zrows).reshape(2 * UP, D)
    t1v = _pass_v(src_u, ig2, is2, zrows).reshape(2 * VP, D)
    y = _scale(t1u, du2, True)
    x = _scale(t1v, dv2, True)
    t2u = _pass_u(x, ig1, is1, zrows).reshape(2 * UP, D)
    t2v = _pass_v(y, ig2, is2, zrows).reshape(2 * VP, D)

    uid = user_ids
    sid = s_item_ids
    tid = t_item_ids
    ids_flat = jnp.stack([
        uid, uid, uid, uid + UP, uid, uid + UP, uid, uid + UP, uid, uid,
        sid, tid, sid, tid + VP, sid, tid + VP, sid, tid + VP,
        sid, tid, sid, tid,
    ]).astype(jnp.int32).reshape(-1)
    tables = (p["s_u_emb"], p["t_u_emb"], t1u, t1u, t2u, t2u, du2, du2,
              f["s_rev"], f["t_rev"],
              p["s_v_emb"], p["t_v_emb"], t1v, t1v, t2v, t2v, dv2, dv2,
              f["s_text"], f["t_text"], f["s_vis"], f["t_vis"])
    gathered = _batch_gather(*tables, ids_flat)

    return _run_head(gathered, p)
